# baseline (speedup 1.0000x reference)
import functools

import jax
import jax.numpy as jnp
from jax import lax
from jax.experimental import pallas as pl
from jax.experimental.pallas import tpu as pltpu
from jax.experimental.pallas import tpu_sc as plsc

N = 3200000
SEG = 100000
Q = 16
NC, NS = 2, 16
NW = NC * NS
B = 128
NBLK = N // B
BLK_LO = NBLK // NW
EXTRA = NBLK - BLK_LO * NW
CB = 4
C = CB * B
NFULL = BLK_LO // CB
PAIRS = (NFULL - 1) // 2
TAIL_LO = BLK_LO - NFULL * CB
TAIL_HI = TAIL_LO + 1
SLAB = 6256
ROWS_LAST = SEG - (NS - 1) * SLAB


def _sc_body(x_hbm, idx_hbm, prm_hbm, out0_hbm, out1_hbm,
             xb0, xb1, ib0, ib1, sb0, sb1, tb0, tb1, pb, tbt, acc,
             sem_in0, sem_in1, sem_sc0, sem_sc1):
    c = lax.axis_index("c")
    s = lax.axis_index("s")
    wid = s * NC + c
    bufs = ((xb0, ib0, sb0, tb0, sem_in0, sem_sc0),
            (xb1, ib1, sb1, tb1, sem_in1, sem_sc1))

    pltpu.sync_copy(prm_hbm, pb.at[pl.ds(0, 4)])
    p0 = pb[0]
    p1 = pb[1]
    s0 = pb[2]
    s1 = pb[3]
    a0 = -(s0 * s0)
    a1 = -(s1 * s1)
    pb[4] = a0
    pb[5] = -2.0 * a0 * p0
    pb[6] = a1
    pb[7] = -2.0 * a1 * p1
    pb[8] = a0 * p0 * p0 + a1 * p1 * p1

    zeros = jnp.zeros((Q,), jnp.float32)

    @plsc.parallel_loop(0, C)
    def _(i):
        tb0[i] = zeros

    slab = pl.multiple_of(s * SLAB, 16)

    def zero_rows(nrows):
        for z in range(nrows // C):
            pltpu.sync_copy(tb0, acc.at[pl.ds(slab + z * C, C)])
        rem = nrows % C
        if rem:
            pltpu.sync_copy(tb0.at[pl.ds(0, rem)],
                            acc.at[pl.ds(slab + (nrows // C) * C, rem)])

    @pl.when(s < NS - 1)
    def _():
        zero_rows(SLAB)

    @pl.when(s == NS - 1)
    def _():
        zero_rows(ROWS_LAST)

    plsc.subcore_barrier()

    blk0 = BLK_LO * wid + jnp.minimum(wid, EXTRA)

    def in_descs(j, b):
        xb, ib, _, _, s_in, _ = bufs[b]
        blk = blk0 + j * CB
        return (
            pltpu.make_async_copy(x_hbm.at[pl.ds(blk, CB)], xb, s_in),
            pltpu.make_async_copy(
                idx_hbm.at[pl.ds(pl.multiple_of(blk * B, B), C)], ib, s_in),
        )

    def sc_descs(b):
        _, _, sb, tb, _, s_sc = bufs[b]
        return [
            pltpu.make_async_copy(tb.at[pl.ds(jj * B, B)],
                                  acc.at[sb.at[pl.ds(jj * B, B)]], s_sc)
            for jj in range(CB)
        ]

    def compute(b, nb, idx_dst):
        xb, ib, sb, tb, _, _ = bufs[b]

        @plsc.parallel_loop(0, nb * 8, unroll=1)
        def _(k):
            g = k // 8
            kk = k % 8
            xv0 = xb[g, 0, pl.ds(16 * kk, 16)]
            xv1 = xb[g, 1, pl.ds(16 * kk, 16)]
            a0 = pb[4]
            b0 = pb[5]
            a1 = pb[6]
            b1 = pb[7]
            cc = pb[8]
            for u in range(16):
                x0v = jnp.broadcast_to(xv0[u], (Q,))
                x1v = jnp.broadcast_to(xv1[u], (Q,))
                t = ((cc + b0 * x0v) + a0 * (x0v * x0v)
                     + (b1 * x1v + a1 * (x1v * x1v)))
                tb[128 * g + 16 * kk + u] = jnp.exp(t)

        if idx_dst is not None:
            @plsc.parallel_loop(0, (nb * B) // 16)
            def _(i):
                idx_dst[pl.ds(16 * i, 16)] = ib[pl.ds(16 * i, 16)]

    for b in (0, 1):
        for d in in_descs(b, b):
            d.start()

    def pair_body(step, _):
        for b in (0, 1):
            j = 2 * step + b

            @pl.when(step > 0)
            def _():
                for d in sc_descs(b):
                    d.wait()

            for d in in_descs(j, b):
                d.wait()
            compute(b, CB, bufs[b][2])
            for d in sc_descs(b):
                d.start(add=True)

            @pl.when(j + 2 < NFULL)
            def _():
                for d in in_descs(j + 2, b):
                    d.start()
        return None

    lax.fori_loop(0, PAIRS, pair_body, None)

    for d in sc_descs(0):
        d.wait()
    for d in in_descs(NFULL - 1, 0):
        d.wait()
    compute(0, CB, bufs[0][2])
    for d in sc_descs(0):
        d.start(add=True)
    for d in sc_descs(1):
        d.wait()
    for d in sc_descs(0):
        d.wait()

    def do_tail(nb):
        n = nb * B
        blk = blk0 + NFULL * CB
        pltpu.sync_copy(x_hbm.at[pl.ds(blk, nb)], xb0.at[pl.ds(0, nb)])
        pltpu.sync_copy(
            idx_hbm.at[pl.ds(pl.multiple_of(blk * B, B), n)],
            ib0.at[pl.ds(0, n)])
        compute(0, nb, None)
        descs = [
            pltpu.make_async_copy(tb0.at[pl.ds(jj * B, B)],
                                  acc.at[ib0.at[pl.ds(jj * B, B)]], sem_sc0)
            for jj in range(nb)
        ]
        for d in descs:
            d.start(add=True)
        for d in descs:
            d.wait()

    @pl.when(wid < EXTRA)
    def _():
        do_tail(TAIL_HI)

    @pl.when(wid >= EXTRA)
    def _():
        do_tail(TAIL_LO)

    plsc.subcore_barrier()
    rowi = lax.iota(jnp.int32, 16)

    def tblock(dst, r0, npts):
        pltpu.sync_copy(acc.at[pl.ds(r0, npts)], tb0.at[pl.ds(0, npts)])
        for grp in range(npts // 16):
            rows = rowi + 16 * grp
            for q in range(Q):
                g = plsc.load_gather(
                    tb0, [rows, jnp.full((16,), q, jnp.int32)])
                tbt[q, pl.ds(16 * grp, 16)] = g
        src = tbt if npts == 128 else tbt.at[:, pl.ds(0, npts)]
        pltpu.sync_copy(src, dst.at[:, pl.ds(pl.multiple_of(r0, 16), npts)])

    def writeback(dst, nrows):
        def tb_body(z, _):
            tblock(dst, slab + z * 128, 128)
            return None

        lax.fori_loop(0, nrows // 128, tb_body, None)
        if nrows % 128:
            tblock(dst, slab + (nrows // 128) * 128, nrows % 128)

    for cid, dst in ((0, out0_hbm), (1, out1_hbm)):
        @pl.when(c == cid)
        def _():
            @pl.when(s < NS - 1)
            def _():
                writeback(dst, SLAB)

            @pl.when(s == NS - 1)
            def _():
                writeback(dst, ROWS_LAST)


_sc_kernel = functools.partial(
    pl.kernel,
    out_type=(jax.ShapeDtypeStruct((Q, SEG), jnp.float32),
              jax.ShapeDtypeStruct((Q, SEG), jnp.float32)),
    mesh=plsc.VectorSubcoreMesh(core_axis_name="c", subcore_axis_name="s"),
    scratch_types=[
        pltpu.VMEM((CB, 2, B), jnp.float32),
        pltpu.VMEM((CB, 2, B), jnp.float32),
        pltpu.VMEM((C,), jnp.int32),
        pltpu.VMEM((C,), jnp.int32),
        pltpu.VMEM((C,), jnp.int32),
        pltpu.VMEM((C,), jnp.int32),
        pltpu.VMEM((C, Q), jnp.float32),
        pltpu.VMEM((C, Q), jnp.float32),
        pltpu.VMEM((9, Q), jnp.float32),
        pltpu.VMEM((Q, 128), jnp.float32),
        pltpu.VMEM_SHARED((SEG, Q), jnp.float32),
        pltpu.SemaphoreType.DMA,
        pltpu.SemaphoreType.DMA,
        pltpu.SemaphoreType.DMA,
        pltpu.SemaphoreType.DMA,
    ],
    compiler_params=pltpu.CompilerParams(use_tc_tiling_on_sc=False,
                                         needs_layout_passes=False),
)(_sc_body)


_BLKC = 12800


def _combine_body(a_ref, b_ref, o_ref):
    o_ref[...] = a_ref[...] + b_ref[...]


def _combine(p0, p1):
    out_t = pl.pallas_call(
        _combine_body,
        grid=(-(-SEG // _BLKC),),
        in_specs=[pl.BlockSpec((Q, _BLKC), lambda i: (0, i)),
                  pl.BlockSpec((Q, _BLKC), lambda i: (0, i))],
        out_specs=pl.BlockSpec((Q, _BLKC), lambda i: (0, i)),
        out_shape=jax.ShapeDtypeStruct((Q, SEG), jnp.float32),
    )(p0, p1)
    return out_t.T


def kernel(input, point_index, sample_points, sample_inverse_sigmas):
    xview = input.reshape(NBLK, B, 2).transpose(0, 2, 1)
    idx = point_index.astype(jnp.int32)
    prm = jnp.concatenate(
        [sample_points.astype(jnp.float32),
         sample_inverse_sigmas.astype(jnp.float32)], axis=0)
    p0, p1 = _sc_kernel(xview, idx, prm)
    return _combine(p0, p1)


# speedup vs baseline: 1.1919x; 1.1919x over previous
import functools

import jax
import jax.numpy as jnp
from jax import lax
from jax.experimental import pallas as pl
from jax.experimental.pallas import tpu as pltpu
from jax.experimental.pallas import tpu_sc as plsc

N = 3200000
SEG = 100000
Q = 16
NC, NS = 2, 16
NW = NC * NS
B = 128
NBLK = N // B
BLK_LO = NBLK // NW
EXTRA = NBLK - BLK_LO * NW
CB = 4
C = CB * B
NFULL = BLK_LO // CB
PAIRS = (NFULL - 1) // 2
TAIL_LO = BLK_LO - NFULL * CB
TAIL_HI = TAIL_LO + 1
SLAB = 6256
ROWS_LAST = SEG - (NS - 1) * SLAB


def _sc_body(x_hbm, idx_hbm, prm_hbm, out0_hbm, out1_hbm,
             xb0, xb1, ib0, ib1, sb0, sb1, tb0, tb1, pb, tbt, acc,
             sem_in0, sem_in1, sem_sc0, sem_sc1):
    c = lax.axis_index("c")
    s = lax.axis_index("s")
    wid = s * NC + c
    bufs = ((xb0, ib0, sb0, tb0, sem_in0, sem_sc0),
            (xb1, ib1, sb1, tb1, sem_in1, sem_sc1))

    pltpu.sync_copy(prm_hbm, pb.at[pl.ds(0, 4)])
    p0 = pb[0]
    p1 = pb[1]
    s0 = pb[2]
    s1 = pb[3]

    zeros = jnp.zeros((Q,), jnp.float32)

    @plsc.parallel_loop(0, C)
    def _(i):
        tb0[i] = zeros

    slab = pl.multiple_of(s * SLAB, 16)

    def zero_rows(nrows):
        for z in range(nrows // C):
            pltpu.sync_copy(tb0, acc.at[pl.ds(slab + z * C, C)])
        rem = nrows % C
        if rem:
            pltpu.sync_copy(tb0.at[pl.ds(0, rem)],
                            acc.at[pl.ds(slab + (nrows // C) * C, rem)])

    @pl.when(s < NS - 1)
    def _():
        zero_rows(SLAB)

    @pl.when(s == NS - 1)
    def _():
        zero_rows(ROWS_LAST)

    plsc.subcore_barrier()

    blk0 = BLK_LO * wid + jnp.minimum(wid, EXTRA)

    def in_descs(j, b):
        xb, ib, _, _, s_in, _ = bufs[b]
        blk = blk0 + j * CB
        return (
            pltpu.make_async_copy(x_hbm.at[pl.ds(blk, CB)], xb, s_in),
            pltpu.make_async_copy(
                idx_hbm.at[pl.ds(pl.multiple_of(blk * B, B), C)], ib, s_in),
        )

    def sc_descs(b):
        _, _, sb, tb, _, s_sc = bufs[b]
        return [
            pltpu.make_async_copy(tb.at[pl.ds(jj * B, B)],
                                  acc.at[sb.at[pl.ds(jj * B, B)]], s_sc)
            for jj in range(CB)
        ]

    def compute(b, nb, idx_dst):
        xb, ib, sb, tb, _, _ = bufs[b]

        @plsc.parallel_loop(0, nb * 8, unroll=2)
        def _(k):
            g = k // 8
            kk = k % 8
            xv0 = xb[g, 0, pl.ds(16 * kk, 16)]
            xv1 = xb[g, 1, pl.ds(16 * kk, 16)]
            for u in range(16):
                z0 = (xv0[u] - p0) * s0
                z1 = (xv1[u] - p1) * s1
                tb[128 * g + 16 * kk + u] = jnp.exp(-(z0 * z0 + z1 * z1))

        if idx_dst is not None:
            @plsc.parallel_loop(0, (nb * B) // 16)
            def _(i):
                idx_dst[pl.ds(16 * i, 16)] = ib[pl.ds(16 * i, 16)]

    for b in (0, 1):
        for d in in_descs(b, b):
            d.start()

    def pair_body(step, _):
        for b in (0, 1):
            j = 2 * step + b

            @pl.when(step > 0)
            def _():
                for d in sc_descs(b):
                    d.wait()

            for d in in_descs(j, b):
                d.wait()
            compute(b, CB, bufs[b][2])
            for d in sc_descs(b):
                d.start(add=True)

            @pl.when(j + 2 < NFULL)
            def _():
                for d in in_descs(j + 2, b):
                    d.start()
        return None

    lax.fori_loop(0, PAIRS, pair_body, None)

    for d in sc_descs(0):
        d.wait()
    for d in in_descs(NFULL - 1, 0):
        d.wait()
    compute(0, CB, bufs[0][2])
    for d in sc_descs(0):
        d.start(add=True)
    for d in sc_descs(1):
        d.wait()
    for d in sc_descs(0):
        d.wait()

    def do_tail(nb):
        n = nb * B
        blk = blk0 + NFULL * CB
        pltpu.sync_copy(x_hbm.at[pl.ds(blk, nb)], xb0.at[pl.ds(0, nb)])
        pltpu.sync_copy(
            idx_hbm.at[pl.ds(pl.multiple_of(blk * B, B), n)],
            ib0.at[pl.ds(0, n)])
        compute(0, nb, None)
        descs = [
            pltpu.make_async_copy(tb0.at[pl.ds(jj * B, B)],
                                  acc.at[ib0.at[pl.ds(jj * B, B)]], sem_sc0)
            for jj in range(nb)
        ]
        for d in descs:
            d.start(add=True)
        for d in descs:
            d.wait()

    @pl.when(wid < EXTRA)
    def _():
        do_tail(TAIL_HI)

    @pl.when(wid >= EXTRA)
    def _():
        do_tail(TAIL_LO)

    plsc.subcore_barrier()
    rowi = lax.iota(jnp.int32, 16)

    def tblock(dst, r0, npts):
        pltpu.sync_copy(acc.at[pl.ds(r0, npts)], tb0.at[pl.ds(0, npts)])
        for grp in range(npts // 16):
            rows = rowi + 16 * grp
            for q in range(Q):
                g = plsc.load_gather(
                    tb0, [rows, jnp.full((16,), q, jnp.int32)])
                tbt[q, pl.ds(16 * grp, 16)] = g
        src = tbt if npts == 128 else tbt.at[:, pl.ds(0, npts)]
        pltpu.sync_copy(src, dst.at[:, pl.ds(pl.multiple_of(r0, 16), npts)])

    def writeback(dst, nrows):
        def tb_body(z, _):
            tblock(dst, slab + z * 128, 128)
            return None

        lax.fori_loop(0, nrows // 128, tb_body, None)
        if nrows % 128:
            tblock(dst, slab + (nrows // 128) * 128, nrows % 128)

    for cid, dst in ((0, out0_hbm), (1, out1_hbm)):
        @pl.when(c == cid)
        def _():
            @pl.when(s < NS - 1)
            def _():
                writeback(dst, SLAB)

            @pl.when(s == NS - 1)
            def _():
                writeback(dst, ROWS_LAST)


_sc_kernel = functools.partial(
    pl.kernel,
    out_type=(jax.ShapeDtypeStruct((Q, SEG), jnp.float32),
              jax.ShapeDtypeStruct((Q, SEG), jnp.float32)),
    mesh=plsc.VectorSubcoreMesh(core_axis_name="c", subcore_axis_name="s"),
    scratch_types=[
        pltpu.VMEM((CB, 2, B), jnp.float32),
        pltpu.VMEM((CB, 2, B), jnp.float32),
        pltpu.VMEM((C,), jnp.int32),
        pltpu.VMEM((C,), jnp.int32),
        pltpu.VMEM((C,), jnp.int32),
        pltpu.VMEM((C,), jnp.int32),
        pltpu.VMEM((C, Q), jnp.float32),
        pltpu.VMEM((C, Q), jnp.float32),
        pltpu.VMEM((9, Q), jnp.float32),
        pltpu.VMEM((Q, 128), jnp.float32),
        pltpu.VMEM_SHARED((SEG, Q), jnp.float32),
        pltpu.SemaphoreType.DMA,
        pltpu.SemaphoreType.DMA,
        pltpu.SemaphoreType.DMA,
        pltpu.SemaphoreType.DMA,
    ],
    compiler_params=pltpu.CompilerParams(use_tc_tiling_on_sc=False,
                                         needs_layout_passes=False),
)(_sc_body)


_BLKC = 12800


def _combine_body(a_ref, b_ref, o_ref):
    o_ref[...] = a_ref[...] + b_ref[...]


def _combine(p0, p1):
    out_t = pl.pallas_call(
        _combine_body,
        grid=(-(-SEG // _BLKC),),
        in_specs=[pl.BlockSpec((Q, _BLKC), lambda i: (0, i)),
                  pl.BlockSpec((Q, _BLKC), lambda i: (0, i))],
        out_specs=pl.BlockSpec((Q, _BLKC), lambda i: (0, i)),
        out_shape=jax.ShapeDtypeStruct((Q, SEG), jnp.float32),
    )(p0, p1)
    return out_t.T


def kernel(input, point_index, sample_points, sample_inverse_sigmas):
    xview = input.reshape(NBLK, B, 2).transpose(0, 2, 1)
    idx = point_index.astype(jnp.int32)
    prm = jnp.concatenate(
        [sample_points.astype(jnp.float32),
         sample_inverse_sigmas.astype(jnp.float32)], axis=0)
    p0, p1 = _sc_kernel(xview, idx, prm)
    return _combine(p0, p1)
